# Initial kernel scaffold; baseline (speedup 1.0000x reference)
import jax
import jax.numpy as jnp
from jax.experimental import pallas as pl


def kernel(x, edge_index, edge_value, target_x, Ws, conv_weight, linear1_w, linear1_b, lin_w, lin_b):
    raise NotImplementedError("write your pallas kernel here")


# speedup vs baseline: 3.5509x; 3.5509x over previous
import functools

import jax
import jax.numpy as jnp
from jax import lax
from jax.experimental import pallas as pl
from jax.experimental.pallas import tpu as pltpu
from jax.experimental.pallas import tpu_sc as plsc

N = 50000
NP = 51200
E = 200000
NUM_ET = 4
L = 2
C = 2
FG = 32
NG = 4
E_PAD = 204800
TE = NUM_ET * E_PAD
EBLK = TE // 128
NSUB = 16
TILE_EDGES = TE // NSUB
CHUNK = 512
NCHUNK = TILE_EDGES // CHUNK
RPT = NP // NSUB
ZR = 64
T_PAD = 4096


def _proj_body(x_ref, wg_ref, out_ref):
    out_ref[0] = jnp.dot(x_ref[...], wg_ref[0],
                         preferred_element_type=jnp.float32)


def _proj(xp, wg):
    bn = 3200
    return pl.pallas_call(
        _proj_body,
        grid=(NG, NP // bn),
        in_specs=[
            pl.BlockSpec((bn, 128), lambda j, nb: (nb, 0)),
            pl.BlockSpec((1, 128, FG), lambda j, nb: (j, 0, 0)),
        ],
        out_specs=pl.BlockSpec((1, bn, FG), lambda j, nb: (j, nb, 0)),
        out_shape=jax.ShapeDtypeStruct((NG, NP, FG), jnp.float32),
    )(xp, wg)


def _edge_scale_body(ev_ref, cw_ref, out_ref):
    et = pl.program_id(0)
    cw = cw_ref[...]
    col = lax.broadcasted_iota(jnp.int32, (8, 128), 1)
    valid = col < NUM_ET
    m = jnp.max(jnp.where(valid, cw, -1e30), axis=1, keepdims=True)
    ex = jnp.where(valid, jnp.exp(cw - m), 0.0)
    filt = ex / jnp.sum(ex, axis=1, keepdims=True)
    onehot = (col == et).astype(jnp.float32)
    fs = jnp.sum(filt * onehot, axis=1, keepdims=True)
    ev = ev_ref[0]
    for l in range(L):
        for ch in range(C):
            out_ref[l, ch, 0] = ev * fs[l * C + ch, 0]


def _edge_scale(ev3, cw8):
    bb = 400
    nb = E_PAD // 128 // bb
    return pl.pallas_call(
        _edge_scale_body,
        grid=(NUM_ET, nb),
        in_specs=[
            pl.BlockSpec((1, bb, 128), lambda et, eb: (et, eb, 0)),
            pl.BlockSpec((8, 128), lambda et, eb: (0, 0)),
        ],
        out_specs=pl.BlockSpec((L, C, 1, bb, 128),
                               lambda et, eb: (0, 0, et, eb, 0)),
        out_shape=jax.ShapeDtypeStruct((L, C, NUM_ET, E_PAD // 128, 128),
                                       jnp.float32),
    )(ev3, cw8)


def _spmm_body(table, rows_h, cols_h, v0_h, v1_h, out_h,
               acc, zbuf, colsv, rowsv, vv, gidx, rowbuf, sem):
    c = lax.axis_index("c")
    s = lax.axis_index("s")

    z16 = jnp.zeros((16,), jnp.float32)

    def _zb(i, carry):
        zbuf[i, pl.ds(0, 16)] = z16
        zbuf[i, pl.ds(16, 16)] = z16
        return carry

    lax.fori_loop(0, ZR, _zb, 0)

    for slot in range(2):
        vh = v0_h if slot == 0 else v1_h
        jbase = (slot * 2) * NP + c * NP

        def _zero(k, carry):
            pltpu.sync_copy(zbuf, acc.at[pl.ds(s * RPT + k * ZR, ZR)])
            return carry

        lax.fori_loop(0, RPT // ZR, _zero, 0)
        plsc.subcore_barrier()

        def _chunk(t, carry):
            blk = s * (TILE_EDGES // 128) + t * (CHUNK // 128)
            pltpu.sync_copy(cols_h.at[pl.ds(blk, 4)], colsv)
            pltpu.sync_copy(rows_h.at[pl.ds(blk, 4)], rowsv)
            pltpu.sync_copy(vh.at[pl.ds(blk, 4)], vv)
            for r in range(4):
                for q in range(8):
                    gidx[r, pl.ds(q * 16, 16)] = (
                        colsv[r, pl.ds(q * 16, 16)] + jbase)
            cps = [
                pltpu.async_copy(table.at[gidx.at[r]],
                                 rowbuf.at[pl.ds(r * 128, 128)], sem)
                for r in range(4)
            ]
            for cp in cps:
                cp.wait()

            def _scale(g, carry2):
                gr = g >> 3
                gq = (g & 7) * 16
                v16 = vv[gr, pl.ds(gq, 16)]
                for e in range(16):
                    bc = lax.gather(
                        v16, jnp.full((16, 1), e, jnp.int32),
                        lax.GatherDimensionNumbers(
                            offset_dims=(), collapsed_slice_dims=(0,),
                            start_index_map=(0,)),
                        (1,), mode=lax.GatherScatterMode.PROMISE_IN_BOUNDS)
                    ea = g * 16 + e
                    rowbuf[ea, pl.ds(0, 16)] = rowbuf[ea, pl.ds(0, 16)] * bc
                    rowbuf[ea, pl.ds(16, 16)] = rowbuf[ea, pl.ds(16, 16)] * bc
                return carry2

            lax.fori_loop(0, CHUNK // 16, _scale, 0)

            cps2 = [
                pltpu.async_copy(rowbuf.at[pl.ds(r * 128, 128)],
                                 acc.at[rowsv.at[r]], sem, add=True)
                for r in range(4)
            ]
            for cp in cps2:
                cp.wait()
            return carry

        lax.fori_loop(0, NCHUNK, _chunk, 0)
        plsc.subcore_barrier()

        pltpu.sync_copy(acc.at[pl.ds(s * RPT, RPT)],
                        out_h.at[pl.ds(jbase + s * RPT, RPT)])
        plsc.subcore_barrier()


def _spmm(table, rows2, cols2, v0, v1):
    mesh = plsc.VectorSubcoreMesh(core_axis_name="c", subcore_axis_name="s")
    f = pl.kernel(
        _spmm_body,
        mesh=mesh,
        compiler_params=pltpu.CompilerParams(use_tc_tiling_on_sc=False),
        out_type=jax.ShapeDtypeStruct((NG * NP, FG), jnp.float32),
        scratch_types=[
            pltpu.VMEM_SHARED((NP, FG), jnp.float32),
            pltpu.VMEM((ZR, FG), jnp.float32),
            pltpu.VMEM((4, 128), jnp.int32),
            pltpu.VMEM((4, 128), jnp.int32),
            pltpu.VMEM((4, 128), jnp.float32),
            pltpu.VMEM((4, 128), jnp.int32),
            pltpu.VMEM((CHUNK, FG), jnp.float32),
            pltpu.SemaphoreType.DMA,
        ],
    )
    return f(table, rows2, cols2, v0, v1)


def _tgt_body(t0, t2, tgt_h, out_h, tgtv, gidx, xv, hv, ov, sem):
    c = lax.axis_index("c")
    s = lax.axis_index("s")
    w = s * 2 + c
    pltpu.sync_copy(tgt_h, tgtv)
    for j in range(NG):
        for q in range(8):
            gidx[0, pl.ds(q * 16, 16)] = tgtv[w, pl.ds(q * 16, 16)] + j * NP
        pltpu.async_copy(t0.at[gidx.at[0]], xv, sem).wait()
        pltpu.async_copy(t2.at[gidx.at[0]], hv, sem).wait()

        def _mix(g, carry):
            r = g >> 1
            q = (g & 1) * 16
            a = xv[r, pl.ds(q, 16)]
            b = hv[r, pl.ds(q, 16)]
            ov[r, pl.ds(q, 16)] = jnp.maximum((a + b) * 0.5, 0.0)
            return carry

        lax.fori_loop(0, 256, _mix, 0)
        pltpu.sync_copy(ov, out_h.at[pl.ds(j * T_PAD + w * 128, 128)])


def _tgt(t0, t2, tgt2):
    mesh = plsc.VectorSubcoreMesh(core_axis_name="c", subcore_axis_name="s")
    f = pl.kernel(
        _tgt_body,
        mesh=mesh,
        compiler_params=pltpu.CompilerParams(use_tc_tiling_on_sc=False),
        out_type=jax.ShapeDtypeStruct((NG * T_PAD, FG), jnp.float32),
        scratch_types=[
            pltpu.VMEM((32, 128), jnp.int32),
            pltpu.VMEM((1, 128), jnp.int32),
            pltpu.VMEM((128, FG), jnp.float32),
            pltpu.VMEM((128, FG), jnp.float32),
            pltpu.VMEM((128, FG), jnp.float32),
            pltpu.SemaphoreType.DMA,
        ],
    )
    return f(t0, t2, tgt2)


def _head_body(hc_ref, w1t_ref, b1_ref, w2t_ref, b2_ref, out_ref):
    h1 = jnp.maximum(
        jnp.dot(hc_ref[...], w1t_ref[...],
                preferred_element_type=jnp.float32) + b1_ref[0:1, :], 0.0)
    out_ref[...] = jnp.dot(h1, w2t_ref[...],
                           preferred_element_type=jnp.float32) + b2_ref[0:1, :]


def _head(hc, w1t, b1p, w2t, b2p):
    return pl.pallas_call(
        _head_body,
        out_shape=jax.ShapeDtypeStruct((T_PAD, 128), jnp.float32),
    )(hc, w1t, b1p, w2t, b2p)


def kernel(x, edge_index, edge_value, target_x, Ws, conv_weight,
           linear1_w, linear1_b, lin_w, lin_b):
    wg = jnp.stack([Ws[j // 2][:, (j % 2) * FG:(j % 2) * FG + FG]
                    for j in range(NG)])
    rows2 = jnp.pad(edge_index[:, 0, :],
                    ((0, 0), (0, E_PAD - E))).reshape(EBLK, 128)
    cols2 = jnp.pad(edge_index[:, 1, :],
                    ((0, 0), (0, E_PAD - E))).reshape(EBLK, 128)
    ev3 = jnp.pad(edge_value,
                  ((0, 0), (0, E_PAD - E))).reshape(NUM_ET, E_PAD // 128, 128)
    cw8 = jnp.pad(conv_weight.reshape(L * C, NUM_ET), ((0, 4), (0, 124)))
    tgt2 = jnp.pad(target_x, (0, T_PAD - target_x.shape[0])).reshape(32, 128)

    xp = jnp.pad(x, ((0, NP - N), (0, 0)))
    h0 = _proj(xp, wg).reshape(NG * NP, FG)
    v = _edge_scale(ev3, cw8).reshape(L, C, EBLK, 128)

    h1 = _spmm(h0, rows2, cols2, v[0, 0], v[0, 1])
    h2 = _spmm(h1, rows2, cols2, v[1, 0], v[1, 1])

    mt = _tgt(h0, h2, tgt2)
    hc = mt.reshape(NG, T_PAD, FG).transpose(1, 0, 2).reshape(T_PAD, 128)

    w1t = linear1_w.T
    b1p = jnp.pad(linear1_b[None, :], ((0, 7), (0, 0)))
    w2t = jnp.pad(lin_w.T, ((0, 0), (0, 128 - lin_w.shape[0])))
    b2p = jnp.pad(lin_b[None, :], ((0, 7), (0, 128 - lin_b.shape[0])))

    y = _head(hc, w1t, b1p, w2t, b2p)
    return y[:target_x.shape[0], :lin_w.shape[0]]


# speedup vs baseline: 5.3572x; 1.5087x over previous
import jax
import jax.numpy as jnp
from jax import lax
from jax.experimental import pallas as pl
from jax.experimental.pallas import tpu as pltpu
from jax.experimental.pallas import tpu_sc as plsc

N = 50000
NP = 51200
E = 200000
NUM_ET = 4
L = 2
C = 2
FG = 32
NG = 4
E_PAD = 204800
TE = NUM_ET * E_PAD
EBLK = TE // 128
NSUB = 16
TILE_EDGES = TE // NSUB
CHUNK = 128
NCH = TILE_EDGES // CHUNK
NB = 4
RPT = NP // NSUB
ZR = 160
T_PAD = 4096


def _proj_body(x_ref, wg_ref, out_ref):
    out_ref[0] = jnp.dot(x_ref[...], wg_ref[0],
                         preferred_element_type=jnp.float32)


def _proj(xp, wg):
    bn = 3200
    return pl.pallas_call(
        _proj_body,
        grid=(NG, NP // bn),
        in_specs=[
            pl.BlockSpec((bn, 128), lambda j, nb: (nb, 0)),
            pl.BlockSpec((1, 128, FG), lambda j, nb: (j, 0, 0)),
        ],
        out_specs=pl.BlockSpec((1, bn, FG), lambda j, nb: (j, nb, 0)),
        out_shape=jax.ShapeDtypeStruct((NG, NP, FG), jnp.float32),
    )(xp, wg)


def _edge_scale_body(ev_ref, cols_ref, rows_ref, cw_ref, out_ref):
    et = pl.program_id(0)
    cw = cw_ref[...]
    col = lax.broadcasted_iota(jnp.int32, (8, 128), 1)
    valid = col < NUM_ET
    m = jnp.max(jnp.where(valid, cw, -1e30), axis=1, keepdims=True)
    ex = jnp.where(valid, jnp.exp(cw - m), 0.0)
    filt = ex / jnp.sum(ex, axis=1, keepdims=True)
    onehot = (col == et).astype(jnp.float32)
    fs = jnp.sum(filt * onehot, axis=1, keepdims=True)
    ev = ev_ref[0]
    for l in range(L):
        out_ref[l, :, 0, :] = cols_ref[0]
        out_ref[l, :, 1, :] = rows_ref[0]
        for ch in range(C):
            out_ref[l, :, 2 + ch, :] = lax.bitcast_convert_type(
                ev * fs[l * C + ch, 0], jnp.int32)


def _edge_scale(ev3, cols3, rows3, cw8):
    bb = 400
    nb = E_PAD // 128 // bb
    espec = pl.BlockSpec((1, bb, 128), lambda et, eb: (et, eb, 0))
    ispec = pl.BlockSpec((1, bb, 128), lambda et, eb: (et, eb, 0))
    return pl.pallas_call(
        _edge_scale_body,
        grid=(NUM_ET, nb),
        in_specs=[
            espec, ispec, ispec,
            pl.BlockSpec((8, 128), lambda et, eb: (0, 0)),
        ],
        out_specs=pl.BlockSpec((L, bb, 4, 128),
                               lambda et, eb: (0, et * nb + eb, 0, 0)),
        out_shape=jax.ShapeDtypeStruct((L, EBLK, 4, 128), jnp.int32),
    )(ev3, cols3, rows3, cw8)


def _spmm_body(table, edges_h, out_h, acc, zbuf,
               ebs, gxs, sxs, vvs, rbs, ess, gss, sss, zsem):
    c = lax.axis_index("c")
    s = lax.axis_index("s")

    z16 = jnp.zeros((16,), jnp.float32)

    def _zb(i, carry):
        zbuf[i, pl.ds(0, 16)] = z16
        zbuf[i, pl.ds(16, 16)] = z16
        return carry

    lax.fori_loop(0, ZR, _zb, 0)

    tile_blk = s * NCH

    def _fire_edge(blk, b):
        pltpu.async_copy(edges_h.at[pl.ds(blk, 1)], ebs[b], ess[b])

    def _wait_edge(b):
        pltpu.make_async_copy(edges_h.at[pl.ds(0, 1)], ebs[b],
                              ess[b]).wait()

    def _prep(b, jbase, slot):
        for q in range(8):
            sl = pl.ds(q * 16, 16)
            gxs[b][0, sl] = ebs[b][0, 0, sl] + jbase
            sxs[b][0, sl] = ebs[b][0, 1, sl]
            vvs[b][0, sl] = plsc.bitcast(ebs[b][0, 2 + slot, sl],
                                         jnp.float32)

    def _fire_gather(b):
        pltpu.async_copy(table.at[gxs[b].at[0]], rbs[b], gss[b])

    def _wait_gather(b):
        pltpu.make_async_copy(table.at[gxs[b].at[0]], rbs[b],
                              gss[b]).wait()

    def _scale(b):
        def _sc(g, carry):
            v16 = vvs[b][0, pl.ds(g * 16, 16)]
            for e in range(16):
                bc = lax.gather(
                    v16, jnp.full((16, 1), e, jnp.int32),
                    lax.GatherDimensionNumbers(
                        offset_dims=(), collapsed_slice_dims=(0,),
                        start_index_map=(0,)),
                    (1,), mode=lax.GatherScatterMode.PROMISE_IN_BOUNDS)
                ea = g * 16 + e
                rbs[b][ea, pl.ds(0, 16)] = rbs[b][ea, pl.ds(0, 16)] * bc
                rbs[b][ea, pl.ds(16, 16)] = rbs[b][ea, pl.ds(16, 16)] * bc
            return carry

        lax.fori_loop(0, CHUNK // 16, _sc, 0)

    def _fire_scatter(b):
        pltpu.async_copy(rbs[b], acc.at[sxs[b].at[0]], sss[b], add=True)

    def _wait_scatter(b):
        pltpu.make_async_copy(rbs[b], acc.at[sxs[b].at[0]],
                              sss[b]).wait()

    for slot in range(2):
        jbase = (slot * 2) * NP + c * NP

        zcps = []
        for k in range(RPT // ZR):
            zcps.append(pltpu.async_copy(
                zbuf, acc.at[pl.ds(s * RPT + k * ZR, ZR)], zsem))
        for cp in zcps:
            cp.wait()
        plsc.subcore_barrier()

        def _back(b1):
            _wait_gather(b1)
            _scale(b1)
            _fire_scatter(b1)

        for b in range(NB):
            _fire_edge(tile_blk + b, b)
        for tb in range(NB):
            _wait_edge(tb)
            _prep(tb, jbase, slot)
            _fire_edge(tile_blk + tb + NB, tb)
            _fire_gather(tb)
            if tb >= NB - 1:
                _back((tb + 1) % NB)

        def _steady(i, carry):
            tb0 = NB + i * NB
            for k in range(NB):
                _wait_edge(k)
                _wait_scatter(k)
                _prep(k, jbase, slot)
                pltpu.async_copy(
                    edges_h.at[pl.ds(tile_blk + tb0 + k + NB, 1)],
                    ebs[k], ess[k])
                _fire_gather(k)
                _back((k + 1) % NB)
            return carry

        lax.fori_loop(0, (NCH - 2 * NB) // NB, _steady, 0)

        for tb in range(NCH - NB, NCH):
            b = tb % NB
            _wait_edge(b)
            _wait_scatter(b)
            _prep(b, jbase, slot)
            _fire_gather(b)
            _back((b + 1) % NB)
        for tb in range(NCH, NCH + NB - 1):
            _back((tb + 1) % NB)
        for b in range(NB):
            _wait_scatter(b)
        plsc.subcore_barrier()

        pltpu.sync_copy(acc.at[pl.ds(s * RPT, RPT)],
                        out_h.at[pl.ds(jbase + s * RPT, RPT)])
        plsc.subcore_barrier()


def _spmm(table, edges_l):
    mesh = plsc.VectorSubcoreMesh(core_axis_name="c", subcore_axis_name="s")
    f = pl.kernel(
        _spmm_body,
        mesh=mesh,
        compiler_params=pltpu.CompilerParams(
            use_tc_tiling_on_sc=False, needs_layout_passes=False),
        out_type=jax.ShapeDtypeStruct((NG * NP, FG), jnp.float32),
        scratch_types=[
            pltpu.VMEM_SHARED((NP, FG), jnp.float32),
            pltpu.VMEM((ZR, FG), jnp.float32),
            [pltpu.VMEM((1, 4, 128), jnp.int32) for _ in range(NB)],
            [pltpu.VMEM((1, 128), jnp.int32) for _ in range(NB)],
            [pltpu.VMEM((1, 128), jnp.int32) for _ in range(NB)],
            [pltpu.VMEM((1, 128), jnp.float32) for _ in range(NB)],
            [pltpu.VMEM((CHUNK, FG), jnp.float32) for _ in range(NB)],
            [pltpu.SemaphoreType.DMA for _ in range(NB)],
            [pltpu.SemaphoreType.DMA for _ in range(NB)],
            [pltpu.SemaphoreType.DMA for _ in range(NB)],
            pltpu.SemaphoreType.DMA,
        ],
    )
    return f(table, edges_l)


def _tgt_body(t0, t2, tgt_h, out_h, tgtv, gidx, xv, hv, ov, sem):
    c = lax.axis_index("c")
    s = lax.axis_index("s")
    w = s * 2 + c
    pltpu.sync_copy(tgt_h, tgtv)
    for j in range(NG):
        for q in range(8):
            gidx[0, pl.ds(q * 16, 16)] = tgtv[w, pl.ds(q * 16, 16)] + j * NP
        cpx = pltpu.async_copy(t0.at[gidx.at[0]], xv, sem)
        cph = pltpu.async_copy(t2.at[gidx.at[0]], hv, sem)
        cpx.wait()
        cph.wait()

        def _mix(g, carry):
            r = g >> 1
            q = (g & 1) * 16
            a = xv[r, pl.ds(q, 16)]
            b = hv[r, pl.ds(q, 16)]
            ov[r, pl.ds(q, 16)] = jnp.maximum((a + b) * 0.5, 0.0)
            return carry

        lax.fori_loop(0, 256, _mix, 0)
        pltpu.sync_copy(ov, out_h.at[pl.ds(j * T_PAD + w * 128, 128)])


def _tgt(t0, t2, tgt2):
    mesh = plsc.VectorSubcoreMesh(core_axis_name="c", subcore_axis_name="s")
    f = pl.kernel(
        _tgt_body,
        mesh=mesh,
        compiler_params=pltpu.CompilerParams(
            use_tc_tiling_on_sc=False, needs_layout_passes=False),
        out_type=jax.ShapeDtypeStruct((NG * T_PAD, FG), jnp.float32),
        scratch_types=[
            pltpu.VMEM((32, 128), jnp.int32),
            pltpu.VMEM((1, 128), jnp.int32),
            pltpu.VMEM((128, FG), jnp.float32),
            pltpu.VMEM((128, FG), jnp.float32),
            pltpu.VMEM((128, FG), jnp.float32),
            pltpu.SemaphoreType.DMA,
        ],
    )
    return f(t0, t2, tgt2)


def _head_body(hc_ref, w1t_ref, b1_ref, w2t_ref, b2_ref, out_ref):
    h1 = jnp.maximum(
        jnp.dot(hc_ref[...], w1t_ref[...],
                preferred_element_type=jnp.float32) + b1_ref[0:1, :], 0.0)
    out_ref[...] = jnp.dot(h1, w2t_ref[...],
                           preferred_element_type=jnp.float32) + b2_ref[0:1, :]


def _head(hc, w1t, b1p, w2t, b2p):
    return pl.pallas_call(
        _head_body,
        out_shape=jax.ShapeDtypeStruct((T_PAD, 128), jnp.float32),
    )(hc, w1t, b1p, w2t, b2p)


def kernel(x, edge_index, edge_value, target_x, Ws, conv_weight,
           linear1_w, linear1_b, lin_w, lin_b):
    wg = jnp.stack([Ws[j // 2][:, (j % 2) * FG:(j % 2) * FG + FG]
                    for j in range(NG)])
    rows3 = jnp.pad(edge_index[:, 0, :],
                    ((0, 0), (0, E_PAD - E))).reshape(NUM_ET, E_PAD // 128,
                                                     128)
    cols3 = jnp.pad(edge_index[:, 1, :],
                    ((0, 0), (0, E_PAD - E))).reshape(NUM_ET, E_PAD // 128,
                                                     128)
    ev3 = jnp.pad(edge_value,
                  ((0, 0), (0, E_PAD - E))).reshape(NUM_ET, E_PAD // 128, 128)
    cw8 = jnp.pad(conv_weight.reshape(L * C, NUM_ET), ((0, 4), (0, 124)))
    tgt2 = jnp.pad(target_x, (0, T_PAD - target_x.shape[0])).reshape(32, 128)

    xp = jnp.pad(x, ((0, NP - N), (0, 0)))
    h0 = _proj(xp, wg).reshape(NG * NP, FG)
    edges = _edge_scale(ev3, cols3, rows3, cw8)

    h1 = _spmm(h0, edges[0])
    h2 = _spmm(h1, edges[1])

    mt = _tgt(h0, h2, tgt2)
    hc = mt.reshape(NG, T_PAD, FG).transpose(1, 0, 2).reshape(T_PAD, 128)

    w1t = linear1_w.T
    b1p = jnp.pad(linear1_b[None, :], ((0, 7), (0, 0)))
    w2t = jnp.pad(lin_w.T, ((0, 0), (0, 128 - lin_w.shape[0])))
    b2p = jnp.pad(lin_b[None, :], ((0, 7), (0, 128 - lin_b.shape[0])))

    y = _head(hc, w1t, b1p, w2t, b2p)
    return y[:target_x.shape[0], :lin_w.shape[0]]


# speedup vs baseline: 5.6539x; 1.0554x over previous
import jax
import jax.numpy as jnp
from jax import lax
from jax.experimental import pallas as pl
from jax.experimental.pallas import tpu as pltpu
from jax.experimental.pallas import tpu_sc as plsc

N = 50000
NP = 50048
E = 200000
NUM_ET = 4
L = 2
C = 2
FG = 32
NG = 4
E_PAD = 204800
TE = NUM_ET * E_PAD
EBLK = TE // 128
NSUB = 16
TILE_EDGES = TE // NSUB
CHUNK = 256
NCH = TILE_EDGES // CHUNK
NB = 3
RPT = NP // NSUB
ZCP = 12
T_PAD = 4096


def _proj_body(x_ref, wg_ref, out_ref):
    out_ref[0] = jnp.dot(x_ref[...], wg_ref[0],
                         preferred_element_type=jnp.float32)


def _proj(xp, wg):
    bn = NP // 16
    return pl.pallas_call(
        _proj_body,
        grid=(NG, NP // bn),
        in_specs=[
            pl.BlockSpec((bn, 128), lambda j, nb: (nb, 0)),
            pl.BlockSpec((1, 128, FG), lambda j, nb: (j, 0, 0)),
        ],
        out_specs=pl.BlockSpec((1, bn, FG), lambda j, nb: (j, nb, 0)),
        out_shape=jax.ShapeDtypeStruct((NG, NP, FG), jnp.float32),
    )(xp, wg)


def _edge_scale_body(ev_ref, cols_ref, rows_ref, cw_ref, out_ref):
    et = pl.program_id(0)
    cw = cw_ref[...]
    col = lax.broadcasted_iota(jnp.int32, (8, 128), 1)
    valid = col < NUM_ET
    m = jnp.max(jnp.where(valid, cw, -1e30), axis=1, keepdims=True)
    ex = jnp.where(valid, jnp.exp(cw - m), 0.0)
    filt = ex / jnp.sum(ex, axis=1, keepdims=True)
    onehot = (col == et).astype(jnp.float32)
    fs = jnp.sum(filt * onehot, axis=1, keepdims=True)
    ev = ev_ref[0]
    for l in range(L):
        out_ref[l, :, 0, :] = cols_ref[0]
        out_ref[l, :, 1, :] = rows_ref[0]
        for ch in range(C):
            out_ref[l, :, 2 + ch, :] = lax.bitcast_convert_type(
                ev * fs[l * C + ch, 0], jnp.int32)


def _edge_scale(ev3, cols3, rows3, cw8):
    bb = 400
    nb = E_PAD // 128 // bb
    espec = pl.BlockSpec((1, bb, 128), lambda et, eb: (et, eb, 0))
    ispec = pl.BlockSpec((1, bb, 128), lambda et, eb: (et, eb, 0))
    return pl.pallas_call(
        _edge_scale_body,
        grid=(NUM_ET, nb),
        in_specs=[
            espec, ispec, ispec,
            pl.BlockSpec((8, 128), lambda et, eb: (0, 0)),
        ],
        out_specs=pl.BlockSpec((L, bb, 4, 128),
                               lambda et, eb: (0, et * nb + eb, 0, 0)),
        out_shape=jax.ShapeDtypeStruct((L, EBLK, 4, 128), jnp.int32),
    )(ev3, cols3, rows3, cw8)


def _spmm_body(table, edges_h, out_h, acc, zbuf,
               ebs, gxs, sxs, vvs, rbs, ess, gss, sss, zsem):
    c = lax.axis_index("c")
    s = lax.axis_index("s")

    z16 = jnp.zeros((16,), jnp.float32)

    def _zb(i, carry):
        rbs[0][i, pl.ds(0, 16)] = z16
        rbs[0][i, pl.ds(16, 16)] = z16
        return carry

    tile_blk = s * (TILE_EDGES // 128)

    def _fire_edge(blk, b):
        pltpu.async_copy(edges_h.at[pl.ds(blk, 2)], ebs[b], ess[b])

    def _wait_edge(b):
        pltpu.make_async_copy(edges_h.at[pl.ds(0, 2)], ebs[b],
                              ess[b]).wait()

    def _prep(b, jbase, slot):
        for r in range(2):
            for q in range(8):
                sl = pl.ds(q * 16, 16)
                gxs[b][r, sl] = ebs[b][r, 0, sl] + jbase
                sxs[b][r, sl] = ebs[b][r, 1, sl]
                vvs[b][r, sl] = plsc.bitcast(ebs[b][r, 2 + slot, sl],
                                             jnp.float32)

    def _fire_gather(b):
        for r in range(2):
            pltpu.async_copy(table.at[gxs[b].at[r]],
                             rbs[b].at[pl.ds(r * 128, 128)], gss[b])

    def _wait_gather(b):
        for r in range(2):
            pltpu.make_async_copy(table.at[gxs[b].at[r]],
                                  rbs[b].at[pl.ds(r * 128, 128)],
                                  gss[b]).wait()

    def _scale(b):
        @plsc.parallel_loop(0, CHUNK // 16)
        def _sc(g):
            v16 = vvs[b][g >> 3, pl.ds((g & 7) * 16, 16)]
            for e in range(16):
                bc = lax.gather(
                    v16, jnp.full((16, 1), e, jnp.int32),
                    lax.GatherDimensionNumbers(
                        offset_dims=(), collapsed_slice_dims=(0,),
                        start_index_map=(0,)),
                    (1,), mode=lax.GatherScatterMode.PROMISE_IN_BOUNDS)
                ea = g * 16 + e
                rbs[b][ea, pl.ds(0, 16)] = rbs[b][ea, pl.ds(0, 16)] * bc
                rbs[b][ea, pl.ds(16, 16)] = rbs[b][ea, pl.ds(16, 16)] * bc

    def _fire_scatter(b):
        for r in range(2):
            pltpu.async_copy(rbs[b].at[pl.ds(r * 128, 128)],
                             acc.at[sxs[b].at[r]], sss[b], add=True)

    def _wait_scatter(b):
        for r in range(2):
            pltpu.make_async_copy(rbs[b].at[pl.ds(r * 128, 128)],
                                  acc.at[sxs[b].at[r]], sss[b]).wait()

    for slot in range(2):
        jbase = (slot * 2) * NP + c * NP

        lax.fori_loop(0, CHUNK, _zb, 0)
        zcps = []
        for k in range(ZCP):
            zcps.append(pltpu.async_copy(
                rbs[0], acc.at[pl.ds(s * RPT + k * CHUNK, CHUNK)], zsem))
        zcps.append(pltpu.async_copy(
            rbs[0].at[pl.ds(0, RPT - ZCP * CHUNK)],
            acc.at[pl.ds(s * RPT + ZCP * CHUNK, RPT - ZCP * CHUNK)], zsem))
        for cp in zcps:
            cp.wait()
        plsc.subcore_barrier()

        def _back(b1):
            _wait_gather(b1)
            _scale(b1)
            _fire_scatter(b1)

        for b in range(NB):
            _fire_edge(tile_blk + b * 2, b)
        for tb in range(NB):
            _wait_edge(tb)
            _prep(tb, jbase, slot)
            _fire_edge(tile_blk + (tb + NB) * 2, tb)
            _fire_gather(tb)
            if tb >= NB - 1:
                _back((tb + 1) % NB)

        n_steady = (NCH - 2 * NB) // NB
        n_extra = (NCH - 2 * NB) - n_steady * NB

        def _steady(i, carry):
            tb0 = NB + i * NB
            for k in range(NB):
                _wait_edge(k)
                _wait_scatter(k)
                _prep(k, jbase, slot)
                pltpu.async_copy(
                    edges_h.at[pl.ds(tile_blk + (tb0 + k + NB) * 2, 2)],
                    ebs[k], ess[k])
                _fire_gather(k)
                _back((k + 1) % NB)
            return carry

        lax.fori_loop(0, n_steady, _steady, 0)

        for tb in range(NB + n_steady * NB,
                        NB + n_steady * NB + n_extra):
            b = tb % NB
            _wait_edge(b)
            _wait_scatter(b)
            _prep(b, jbase, slot)
            _fire_edge(tile_blk + (tb + NB) * 2, b)
            _fire_gather(b)
            _back((b + 1) % NB)

        for tb in range(NCH - NB, NCH):
            b = tb % NB
            _wait_edge(b)
            _wait_scatter(b)
            _prep(b, jbase, slot)
            _fire_gather(b)
            _back((b + 1) % NB)
        for tb in range(NCH, NCH + NB - 1):
            _back((tb + 1) % NB)
        for b in range(NB):
            _wait_scatter(b)
        plsc.subcore_barrier()

        pltpu.sync_copy(acc.at[pl.ds(s * RPT, RPT)],
                        out_h.at[pl.ds(jbase + s * RPT, RPT)])
        plsc.subcore_barrier()


def _spmm(table, edges_l):
    mesh = plsc.VectorSubcoreMesh(core_axis_name="c", subcore_axis_name="s")
    f = pl.kernel(
        _spmm_body,
        mesh=mesh,
        compiler_params=pltpu.CompilerParams(
            use_tc_tiling_on_sc=False, needs_layout_passes=False),
        out_type=jax.ShapeDtypeStruct((NG * NP, FG), jnp.float32),
        scratch_types=[
            pltpu.VMEM_SHARED((NP, FG), jnp.float32),
            pltpu.VMEM((8, FG), jnp.float32),
            [pltpu.VMEM((2, 4, 128), jnp.int32) for _ in range(NB)],
            [pltpu.VMEM((2, 128), jnp.int32) for _ in range(NB)],
            [pltpu.VMEM((2, 128), jnp.int32) for _ in range(NB)],
            [pltpu.VMEM((2, 128), jnp.float32) for _ in range(NB)],
            [pltpu.VMEM((CHUNK, FG), jnp.float32) for _ in range(NB)],
            [pltpu.SemaphoreType.DMA for _ in range(NB)],
            [pltpu.SemaphoreType.DMA for _ in range(NB)],
            [pltpu.SemaphoreType.DMA for _ in range(NB)],
            pltpu.SemaphoreType.DMA,
        ],
    )
    return f(table, edges_l)


def _tgt_body(t0, t2, tgt_h, out_h, tgtv, gidx, xv, hv, ov, sem):
    c = lax.axis_index("c")
    s = lax.axis_index("s")
    w = s * 2 + c
    pltpu.sync_copy(tgt_h, tgtv)
    cps = []
    for j in range(NG):
        for q in range(8):
            gidx[j, pl.ds(q * 16, 16)] = tgtv[w, pl.ds(q * 16, 16)] + j * NP
        cps.append(pltpu.async_copy(t0.at[gidx.at[j]],
                                    xv.at[pl.ds(j * 128, 128)], sem))
        cps.append(pltpu.async_copy(t2.at[gidx.at[j]],
                                    hv.at[pl.ds(j * 128, 128)], sem))
    for cp in cps:
        cp.wait()

    def _mix(g, carry):
        r = g >> 1
        q = (g & 1) * 16
        a = xv[r, pl.ds(q, 16)]
        b = hv[r, pl.ds(q, 16)]
        ov[r, pl.ds(q, 16)] = jnp.maximum((a + b) * 0.5, 0.0)
        return carry

    lax.fori_loop(0, 1024, _mix, 0)
    for j in range(NG):
        pltpu.sync_copy(ov.at[pl.ds(j * 128, 128)],
                        out_h.at[pl.ds(j * T_PAD + w * 128, 128)])


def _tgt(t0, t2, tgt2):
    mesh = plsc.VectorSubcoreMesh(core_axis_name="c", subcore_axis_name="s")
    f = pl.kernel(
        _tgt_body,
        mesh=mesh,
        compiler_params=pltpu.CompilerParams(
            use_tc_tiling_on_sc=False, needs_layout_passes=False),
        out_type=jax.ShapeDtypeStruct((NG * T_PAD, FG), jnp.float32),
        scratch_types=[
            pltpu.VMEM((32, 128), jnp.int32),
            pltpu.VMEM((NG, 128), jnp.int32),
            pltpu.VMEM((NG * 128, FG), jnp.float32),
            pltpu.VMEM((NG * 128, FG), jnp.float32),
            pltpu.VMEM((NG * 128, FG), jnp.float32),
            pltpu.SemaphoreType.DMA,
        ],
    )
    return f(t0, t2, tgt2)


def _head_body(hc_ref, w1t_ref, b1_ref, w2t_ref, b2_ref, out_ref):
    h1 = jnp.maximum(
        jnp.dot(hc_ref[...], w1t_ref[...],
                preferred_element_type=jnp.float32) + b1_ref[0:1, :], 0.0)
    out_ref[...] = jnp.dot(h1, w2t_ref[...],
                           preferred_element_type=jnp.float32) + b2_ref[0:1, :]


def _head(hc, w1t, b1p, w2t, b2p):
    return pl.pallas_call(
        _head_body,
        out_shape=jax.ShapeDtypeStruct((T_PAD, 128), jnp.float32),
    )(hc, w1t, b1p, w2t, b2p)


def kernel(x, edge_index, edge_value, target_x, Ws, conv_weight,
           linear1_w, linear1_b, lin_w, lin_b):
    wg = jnp.stack([Ws[j // 2][:, (j % 2) * FG:(j % 2) * FG + FG]
                    for j in range(NG)])
    rows3 = jnp.pad(edge_index[:, 0, :],
                    ((0, 0), (0, E_PAD - E))).reshape(NUM_ET, E_PAD // 128,
                                                     128)
    cols3 = jnp.pad(edge_index[:, 1, :],
                    ((0, 0), (0, E_PAD - E))).reshape(NUM_ET, E_PAD // 128,
                                                     128)
    ev3 = jnp.pad(edge_value,
                  ((0, 0), (0, E_PAD - E))).reshape(NUM_ET, E_PAD // 128, 128)
    cw8 = jnp.pad(conv_weight.reshape(L * C, NUM_ET), ((0, 4), (0, 124)))
    tgt2 = jnp.pad(target_x, (0, T_PAD - target_x.shape[0])).reshape(32, 128)

    xp = jnp.pad(x, ((0, NP - N), (0, 0)))
    h0 = _proj(xp, wg).reshape(NG * NP, FG)
    edges = _edge_scale(ev3, cols3, rows3, cw8)

    h1 = _spmm(h0, edges[0])
    h2 = _spmm(h1, edges[1])

    mt = _tgt(h0, h2, tgt2)
    hc = mt.reshape(NG, T_PAD, FG).transpose(1, 0, 2).reshape(T_PAD, 128)

    w1t = linear1_w.T
    b1p = jnp.pad(linear1_b[None, :], ((0, 7), (0, 0)))
    w2t = jnp.pad(lin_w.T, ((0, 0), (0, 128 - lin_w.shape[0])))
    b2p = jnp.pad(lin_b[None, :], ((0, 7), (0, 128 - lin_b.shape[0])))

    y = _head(hc, w1t, b1p, w2t, b2p)
    return y[:target_x.shape[0], :lin_w.shape[0]]
